# baseline (speedup 1.0000x reference)
import jax
import jax.numpy as jnp
from jax.experimental import pallas as pl
from jax.experimental.pallas import tpu as pltpu

T = 8
DIM = 1024
INTER = 2816
E = 8
TOPK = 2

BI = 1408
NJ = INTER // BI


def _ffn_kernel(order_ref, valid_ref, ew_ref, idx_ref, x_ref,
                w1_ref, w3_ref, w2_ref, out_ref):
    i = pl.program_id(1)

    @pl.when(i == 0)
    def _init():
        out_ref[...] = jnp.zeros_like(out_ref)

    e = order_ref[i]
    v = valid_ref[i].astype(jnp.float32)

    ew = ew_ref[...]
    ewn = ew / jnp.sum(ew, axis=-1, keepdims=True)
    idx = idx_ref[...]
    scale = v * jnp.sum(jnp.where(idx == e, ewn, 0.0), axis=-1, keepdims=True)

    x = x_ref[...]
    w1 = w1_ref[0]
    w3 = w3_ref[0]
    w2 = w2_ref[0]

    h1 = jax.lax.dot_general(x, w1, (((1,), (1,)), ((), ())),
                             preferred_element_type=jnp.float32)
    h3 = jax.lax.dot_general(x, w3, (((1,), (1,)), ((), ())),
                             preferred_element_type=jnp.float32)
    h = (h1 * jax.nn.sigmoid(h1)) * h3 * scale

    contrib = jax.lax.dot_general(h, w2, (((1,), (1,)), ((), ())),
                                  preferred_element_type=jnp.float32)
    out_ref[0] += contrib


def kernel(x, expert_weights, expert_indices, w1, w2, w3):
    idx = expert_indices.astype(jnp.int32)

    used = jnp.zeros((E,), jnp.int32).at[idx.reshape(-1)].set(1, mode="drop")
    sorted_experts = jnp.argsort(-used, stable=True).astype(jnp.int32)
    num_used = jnp.sum(used)
    slot = jnp.arange(E, dtype=jnp.int32)
    order = sorted_experts[jnp.minimum(slot, num_used - 1)]
    valid = (slot < num_used).astype(jnp.int32)

    grid_spec = pltpu.PrefetchScalarGridSpec(
        num_scalar_prefetch=2,
        grid=(NJ, num_used),
        in_specs=[
            pl.BlockSpec((T, TOPK), lambda j, i, order, valid: (0, 0)),
            pl.BlockSpec((T, TOPK), lambda j, i, order, valid: (0, 0)),
            pl.BlockSpec((T, DIM), lambda j, i, order, valid: (0, 0)),
            pl.BlockSpec((1, BI, DIM), lambda j, i, order, valid: (order[i], j, 0)),
            pl.BlockSpec((1, BI, DIM), lambda j, i, order, valid: (order[i], j, 0)),
            pl.BlockSpec((1, DIM, BI), lambda j, i, order, valid: (order[i], 0, j)),
        ],
        out_specs=pl.BlockSpec((1, T, DIM), lambda j, i, order, valid: (j, 0, 0)),
    )
    partial = pl.pallas_call(
        _ffn_kernel,
        grid_spec=grid_spec,
        out_shape=jax.ShapeDtypeStruct((NJ, T, DIM), jnp.float32),
        compiler_params=pltpu.CompilerParams(
            dimension_semantics=("parallel", "arbitrary"),
        ),
    )(order, valid, expert_weights, idx, x, w1, w3, w2)
    return jnp.sum(partial, axis=0)


# speedup vs baseline: 1.0535x; 1.0535x over previous
import jax
import jax.numpy as jnp
from jax.experimental import pallas as pl
from jax.experimental.pallas import tpu as pltpu

T = 8
DIM = 1024
INTER = 2816
E = 8
TOPK = 2

BI = 1408
NJ = INTER // BI


def _ffn_kernel(order_ref, ew_ref, idx_ref, x_ref,
                w1_ref, w3_ref, w2_ref, out_ref):
    j = pl.program_id(0)
    i = pl.program_id(1)

    @pl.when(jnp.logical_and(i == 0, j == 0))
    def _init():
        out_ref[...] = jnp.zeros_like(out_ref)

    e = order_ref[i]

    ew = ew_ref[...]
    ewn = ew / jnp.sum(ew, axis=-1, keepdims=True)
    idx = idx_ref[...]
    scale = jnp.sum(jnp.where(idx == e, ewn, 0.0), axis=-1, keepdims=True)

    x = x_ref[...]
    w1 = w1_ref[0]
    w3 = w3_ref[0]
    w2 = w2_ref[0]

    h1 = jax.lax.dot_general(x, w1, (((1,), (1,)), ((), ())),
                             preferred_element_type=jnp.float32)
    h3 = jax.lax.dot_general(x, w3, (((1,), (1,)), ((), ())),
                             preferred_element_type=jnp.float32)
    h = (h1 * jax.nn.sigmoid(h1)) * h3 * scale

    contrib = jax.lax.dot_general(h, w2, (((1,), (1,)), ((), ())),
                                  preferred_element_type=jnp.float32)
    out_ref[...] += contrib


def kernel(x, expert_weights, expert_indices, w1, w2, w3):
    idx = expert_indices.astype(jnp.int32)

    used = jnp.zeros((E,), jnp.int32).at[idx.reshape(-1)].set(1, mode="drop")
    order = jnp.argsort(-used, stable=True).astype(jnp.int32)
    num_used = jnp.sum(used)

    grid_spec = pltpu.PrefetchScalarGridSpec(
        num_scalar_prefetch=1,
        grid=(NJ, num_used),
        in_specs=[
            pl.BlockSpec((T, TOPK), lambda j, i, order: (0, 0)),
            pl.BlockSpec((T, TOPK), lambda j, i, order: (0, 0)),
            pl.BlockSpec((T, DIM), lambda j, i, order: (0, 0)),
            pl.BlockSpec((1, BI, DIM), lambda j, i, order: (order[i], j, 0)),
            pl.BlockSpec((1, BI, DIM), lambda j, i, order: (order[i], j, 0)),
            pl.BlockSpec((1, DIM, BI), lambda j, i, order: (order[i], 0, j)),
        ],
        out_specs=pl.BlockSpec((T, DIM), lambda j, i, order: (0, 0)),
    )
    return pl.pallas_call(
        _ffn_kernel,
        grid_spec=grid_spec,
        out_shape=jax.ShapeDtypeStruct((T, DIM), jnp.float32),
    )(order, expert_weights, idx, x, w1, w3, w2)


# speedup vs baseline: 1.0554x; 1.0017x over previous
import jax
import jax.numpy as jnp
from jax.experimental import pallas as pl
from jax.experimental.pallas import tpu as pltpu

T = 8
DIM = 1024
INTER = 2816
E = 8
TOPK = 2

BI = 1408
NJ = INTER // BI


def _ffn_kernel(order_ref, ew_ref, idx_ref, x_ref,
                w1_ref, w3_ref, w2_ref, out_ref):
    i = pl.program_id(0)
    j = pl.program_id(1)

    @pl.when(jnp.logical_and(i == 0, j == 0))
    def _init():
        out_ref[...] = jnp.zeros_like(out_ref)

    e = order_ref[i]

    ew = ew_ref[...]
    ewn = ew / jnp.sum(ew, axis=-1, keepdims=True)
    idx = idx_ref[...]
    scale = jnp.sum(jnp.where(idx == e, ewn, 0.0), axis=-1, keepdims=True)

    x = x_ref[...]
    w1 = w1_ref[0]
    w3 = w3_ref[0]
    w2 = w2_ref[0]

    h1 = jax.lax.dot_general(x, w1, (((1,), (1,)), ((), ())),
                             preferred_element_type=jnp.float32)
    h3 = jax.lax.dot_general(x, w3, (((1,), (1,)), ((), ())),
                             preferred_element_type=jnp.float32)
    h = (h1 * jax.nn.sigmoid(h1)) * h3 * scale

    contrib = jax.lax.dot_general(h, w2, (((1,), (1,)), ((), ())),
                                  preferred_element_type=jnp.float32)
    out_ref[...] += contrib


def kernel(x, expert_weights, expert_indices, w1, w2, w3):
    idx = expert_indices.astype(jnp.int32)

    used = jnp.zeros((E,), jnp.int32).at[idx.reshape(-1)].set(1, mode="drop")
    order = jnp.argsort(-used, stable=True).astype(jnp.int32)
    num_used = jnp.sum(used)

    grid_spec = pltpu.PrefetchScalarGridSpec(
        num_scalar_prefetch=1,
        grid=(num_used, NJ),
        in_specs=[
            pl.BlockSpec((T, TOPK), lambda i, j, order: (0, 0)),
            pl.BlockSpec((T, TOPK), lambda i, j, order: (0, 0)),
            pl.BlockSpec((T, DIM), lambda i, j, order: (0, 0)),
            pl.BlockSpec((1, BI, DIM), lambda i, j, order: (order[i], j, 0)),
            pl.BlockSpec((1, BI, DIM), lambda i, j, order: (order[i], j, 0)),
            pl.BlockSpec((1, DIM, BI), lambda i, j, order: (order[i], 0, j)),
        ],
        out_specs=pl.BlockSpec((T, DIM), lambda i, j, order: (0, 0)),
    )
    return pl.pallas_call(
        _ffn_kernel,
        grid_spec=grid_spec,
        out_shape=jax.ShapeDtypeStruct((T, DIM), jnp.float32),
    )(order, expert_weights, idx, x, w1, w3, w2)
